# baseline (speedup 1.0000x reference)
import functools

import jax
import jax.numpy as jnp
from jax import lax
from jax.experimental import pallas as pl
from jax.experimental.pallas import tpu as pltpu
from jax.experimental.pallas import tpu_sc as plsc

_NUM_FIELDS = 26
_FIELD_DIM = 100000
_EMBED_DIM = 16
_VOCAB = _NUM_FIELDS * _FIELD_DIM

_info = plsc.get_sparse_core_info()
_NC, _NS = _info.num_cores, _info.num_subcores
_NW = _NC * _NS

_NBLK = _VOCAB // 128
_REM = _VOCAB - _NBLK * 128
_PAIRS = (_NBLK // _NW + 2) // 2


def _shuffle(s_v, o_v, lane, n):
  for i in range(n):
    o_v[pl.ds(i * 16, 16)] = plsc.load_gather(
        s_v, [lane, jnp.full((16,), i, jnp.int32)])


def _make_retile():
  mesh = plsc.VectorSubcoreMesh(core_axis_name="c", subcore_axis_name="s")

  @functools.partial(
      pl.kernel,
      mesh=mesh,
      out_type=jax.ShapeDtypeStruct((_VOCAB * _EMBED_DIM,), jnp.float32),
      scratch_types=[
          pltpu.VMEM((_EMBED_DIM, 128), jnp.float32),
          pltpu.VMEM((_EMBED_DIM, 128), jnp.float32),
          pltpu.VMEM((128 * _EMBED_DIM,), jnp.float32),
          pltpu.VMEM((128 * _EMBED_DIM,), jnp.float32),
          pltpu.SemaphoreType.DMA,
          pltpu.SemaphoreType.DMA,
          pltpu.SemaphoreType.DMA,
          pltpu.SemaphoreType.DMA,
      ],
      compiler_params=pltpu.CompilerParams(needs_layout_passes=False),
  )
  def k(wt_hbm, rem_hbm, out_hbm, s_a, s_b, o_a, o_b, ia, ib, oa, ob):
    wid = lax.axis_index("s") * _NC + lax.axis_index("c")
    lane = jax.lax.broadcasted_iota(jnp.int32, (16,), 0)

    def body(it, _):
      b1 = wid + (2 * it) * _NW
      b2 = wid + (2 * it + 1) * _NW
      v1 = b1 < _NBLK
      v2 = b2 < _NBLK

      @pl.when(v1)
      def _():
        pltpu.async_copy(wt_hbm.at[:, pl.ds(b1 * 128, 128)], s_a, ia)

      @pl.when(v2)
      def _():
        pltpu.async_copy(wt_hbm.at[:, pl.ds(b2 * 128, 128)], s_b, ib)

      @pl.when(v1)
      def _():
        pltpu.make_async_copy(wt_hbm.at[:, pl.ds(b1 * 128, 128)], s_a,
                              ia).wait()
        _shuffle(s_a, o_a, lane, 128)
        pltpu.async_copy(o_a, out_hbm.at[pl.ds(b1 * 2048, 2048)], oa).wait()

      @pl.when(v2)
      def _():
        pltpu.make_async_copy(wt_hbm.at[:, pl.ds(b2 * 128, 128)], s_b,
                              ib).wait()
        _shuffle(s_b, o_b, lane, 128)
        pltpu.async_copy(o_b, out_hbm.at[pl.ds(b2 * 2048, 2048)], ob).wait()

      return ()

    lax.fori_loop(0, _PAIRS, body, (), unroll=False)

    @pl.when(wid == _NW - 1)
    def _():
      pltpu.sync_copy(rem_hbm, o_a.at[pl.ds(0, _REM * 16)])
      pltpu.sync_copy(o_a.at[pl.ds(0, _REM * 16)],
                      out_hbm.at[pl.ds(_NBLK * 2048, _REM * 16)])

  return k


def _make_lookup(batch: int):
  assert batch % _NW == 0
  bpw = batch // _NW
  mesh = plsc.VectorSubcoreMesh(core_axis_name="c", subcore_axis_name="s")
  d = _EMBED_DIM

  @functools.partial(
      pl.kernel,
      mesh=mesh,
      out_type=jax.ShapeDtypeStruct((_NUM_FIELDS, d, batch), jnp.float32),
      scratch_types=[
          pltpu.VMEM((bpw,), jnp.int32),
          pltpu.VMEM((bpw, d), jnp.float32),
          pltpu.VMEM((d, bpw), jnp.float32),
          pltpu.SemaphoreType.DMA,
      ],
      compiler_params=pltpu.CompilerParams(use_tc_tiling_on_sc=False,
                                           needs_layout_passes=False),
  )
  def k(idx_hbm, table_hbm, out_hbm, idx_v, rows_v, cols_v, gsem):
    wid = lax.axis_index("s") * _NC + lax.axis_index("c")
    b0 = wid * bpw
    lane = jax.lax.broadcasted_iota(jnp.int32, (16,), 0)

    def body(f, _):
      pltpu.sync_copy(idx_hbm.at[f, pl.ds(b0, bpw)], idx_v)
      pltpu.async_copy(table_hbm.at[idx_v], rows_v, gsem).wait()
      def col(r16, _):
        rows = r16 * 16 + lane
        for c in range(d):
          cols_v[c, pl.ds(r16 * 16, 16)] = plsc.load_gather(
              rows_v, [rows, jnp.full((16,), c, jnp.int32)])
        return ()

      lax.fori_loop(0, bpw // 16, col, (), unroll=False)
      pltpu.sync_copy(cols_v, out_hbm.at[f, :, pl.ds(b0, bpw)])
      return ()

    lax.fori_loop(0, _NUM_FIELDS, body, (), unroll=False)

  return k


def kernel(x, weight):
  b, f = x.shape
  offsets = jnp.arange(f, dtype=jnp.int32) * _FIELD_DIM
  idx_t = x.T + offsets[:, None]
  rem_flat = weight[_NBLK * 128:, :].reshape(-1)
  table_flat = _make_retile()(weight.T, rem_flat)
  table = table_flat.reshape(_VOCAB, _EMBED_DIM)
  out_t = _make_lookup(b)(idx_t, table)
  return out_t.transpose(2, 0, 1)


# speedup vs baseline: 1.1019x; 1.1019x over previous
import functools

import jax
import jax.numpy as jnp
from jax import lax
from jax.experimental import pallas as pl
from jax.experimental.pallas import tpu as pltpu
from jax.experimental.pallas import tpu_sc as plsc

_NUM_FIELDS = 26
_FIELD_DIM = 100000
_EMBED_DIM = 16
_VOCAB = _NUM_FIELDS * _FIELD_DIM

_info = plsc.get_sparse_core_info()
_NC, _NS = _info.num_cores, _info.num_subcores
_NW = _NC * _NS

_NBLK = _VOCAB // 128
_REM = _VOCAB - _NBLK * 128
_PAIRS = -(-_NBLK // (2 * _NW))


def _shuffle(s_v, o_v, lane, n):
  for i in range(n):
    o_v[pl.ds(i * 16, 16)] = plsc.load_gather(
        s_v, [lane, jnp.full((16,), i, jnp.int32)])


def _make_retile():
  mesh = plsc.VectorSubcoreMesh(core_axis_name="c", subcore_axis_name="s")

  @functools.partial(
      pl.kernel,
      mesh=mesh,
      out_type=jax.ShapeDtypeStruct((_VOCAB * _EMBED_DIM,), jnp.float32),
      scratch_types=[
          pltpu.VMEM((_EMBED_DIM, 128), jnp.float32),
          pltpu.VMEM((_EMBED_DIM, 128), jnp.float32),
          pltpu.VMEM((128 * _EMBED_DIM,), jnp.float32),
          pltpu.VMEM((128 * _EMBED_DIM,), jnp.float32),
          pltpu.SemaphoreType.DMA,
          pltpu.SemaphoreType.DMA,
          pltpu.SemaphoreType.DMA,
          pltpu.SemaphoreType.DMA,
      ],
      compiler_params=pltpu.CompilerParams(needs_layout_passes=False),
  )
  def k(wt_hbm, rem_hbm, out_hbm, s_a, s_b, o_a, o_b, ia, ib, oa, ob):
    wid = lax.axis_index("s") * _NC + lax.axis_index("c")
    lane = jax.lax.broadcasted_iota(jnp.int32, (16,), 0)

    def blk(step):
      b = wid + step * _NW
      return jnp.where(b < _NBLK, b, b - _NBLK)

    def start_in(b, s_v, sem):
      pltpu.async_copy(wt_hbm.at[:, pl.ds(b * 128, 128)], s_v, sem)

    def wait_in(b, s_v, sem):
      pltpu.make_async_copy(wt_hbm.at[:, pl.ds(b * 128, 128)], s_v,
                            sem).wait()

    def drain_out(o_v, sem):
      pltpu.make_async_copy(o_v, out_hbm.at[pl.ds(0, 2048)], sem).wait()

    start_in(blk(0), s_a, ia)
    start_in(blk(1), s_b, ib)

    def body(it, _):
      b1 = blk(2 * it)
      b2 = blk(2 * it + 1)

      @pl.when(it > 0)
      def _():
        drain_out(o_a, oa)

      wait_in(b1, s_a, ia)
      _shuffle(s_a, o_a, lane, 128)
      pltpu.async_copy(o_a, out_hbm.at[pl.ds(b1 * 2048, 2048)], oa)

      @pl.when(it < _PAIRS - 1)
      def _():
        start_in(blk(2 * it + 2), s_a, ia)

      @pl.when(it > 0)
      def _():
        drain_out(o_b, ob)

      wait_in(b2, s_b, ib)
      _shuffle(s_b, o_b, lane, 128)
      pltpu.async_copy(o_b, out_hbm.at[pl.ds(b2 * 2048, 2048)], ob)

      @pl.when(it < _PAIRS - 1)
      def _():
        start_in(blk(2 * it + 3), s_b, ib)

      return ()

    lax.fori_loop(0, _PAIRS, body, (), unroll=False)
    drain_out(o_a, oa)
    drain_out(o_b, ob)

    @pl.when(wid == _NW - 1)
    def _():
      pltpu.sync_copy(rem_hbm, o_a.at[pl.ds(0, _REM * 16)])
      pltpu.sync_copy(o_a.at[pl.ds(0, _REM * 16)],
                      out_hbm.at[pl.ds(_NBLK * 2048, _REM * 16)])

  return k


def _make_lookup(batch: int):
  assert batch % _NW == 0
  bpw = batch // _NW
  mesh = plsc.VectorSubcoreMesh(core_axis_name="c", subcore_axis_name="s")
  d = _EMBED_DIM

  @functools.partial(
      pl.kernel,
      mesh=mesh,
      out_type=jax.ShapeDtypeStruct((_NUM_FIELDS, d, batch), jnp.float32),
      scratch_types=[
          pltpu.VMEM((bpw,), jnp.int32),
          pltpu.VMEM((bpw, d), jnp.float32),
          pltpu.VMEM((d, bpw), jnp.float32),
          pltpu.SemaphoreType.DMA,
      ],
      compiler_params=pltpu.CompilerParams(use_tc_tiling_on_sc=False,
                                           needs_layout_passes=False),
  )
  def k(idx_hbm, table_hbm, out_hbm, idx_v, rows_v, cols_v, gsem):
    wid = lax.axis_index("s") * _NC + lax.axis_index("c")
    b0 = wid * bpw
    lane = jax.lax.broadcasted_iota(jnp.int32, (16,), 0)

    def body(f, _):
      pltpu.sync_copy(idx_hbm.at[f, pl.ds(b0, bpw)], idx_v)
      pltpu.async_copy(table_hbm.at[idx_v], rows_v, gsem).wait()
      def col(r16, _):
        rows = r16 * 16 + lane
        for c in range(d):
          cols_v[c, pl.ds(r16 * 16, 16)] = plsc.load_gather(
              rows_v, [rows, jnp.full((16,), c, jnp.int32)])
        return ()

      lax.fori_loop(0, bpw // 16, col, (), unroll=False)
      pltpu.sync_copy(cols_v, out_hbm.at[f, :, pl.ds(b0, bpw)])
      return ()

    lax.fori_loop(0, _NUM_FIELDS, body, (), unroll=False)

  return k


def kernel(x, weight):
  b, f = x.shape
  offsets = jnp.arange(f, dtype=jnp.int32) * _FIELD_DIM
  idx_t = x.T + offsets[:, None]
  rem_flat = weight[_NBLK * 128:, :].reshape(-1)
  table_flat = _make_retile()(weight.T, rem_flat)
  table = table_flat.reshape(_VOCAB, _EMBED_DIM)
  out_t = _make_lookup(b)(idx_t, table)
  return out_t.transpose(2, 0, 1)


# speedup vs baseline: 2.6090x; 2.3677x over previous
import functools

import jax
import jax.numpy as jnp
from jax import lax
from jax.experimental import pallas as pl
from jax.experimental.pallas import tpu as pltpu
from jax.experimental.pallas import tpu_sc as plsc

_NUM_FIELDS = 26
_FIELD_DIM = 100000
_EMBED_DIM = 16
_VOCAB = _NUM_FIELDS * _FIELD_DIM

_info = plsc.get_sparse_core_info()
_NC, _NS = _info.num_cores, _info.num_subcores
_NW = _NC * _NS

_NBLK = _VOCAB // 128
_REM = _VOCAB - _NBLK * 128
_PAIRS = -(-_NBLK // (2 * _NW))


def _shuffle(s_v, o_v, lane, n):
  def step(i16, _):
    i = i16 * 16 + lane
    for j in range(16):
      d = (j + lane) & 15
      val = plsc.load_gather(s_v, [d, i])
      plsc.store_scatter(o_v, [i * 16 + d], val)
    return ()

  lax.fori_loop(0, n // 16, step, (), unroll=False)


def _make_retile():
  mesh = plsc.VectorSubcoreMesh(core_axis_name="c", subcore_axis_name="s")

  @functools.partial(
      pl.kernel,
      mesh=mesh,
      out_type=jax.ShapeDtypeStruct((_VOCAB * _EMBED_DIM,), jnp.float32),
      scratch_types=[
          pltpu.VMEM((_EMBED_DIM, 128), jnp.float32),
          pltpu.VMEM((_EMBED_DIM, 128), jnp.float32),
          pltpu.VMEM((128 * _EMBED_DIM,), jnp.float32),
          pltpu.VMEM((128 * _EMBED_DIM,), jnp.float32),
          pltpu.SemaphoreType.DMA,
          pltpu.SemaphoreType.DMA,
          pltpu.SemaphoreType.DMA,
          pltpu.SemaphoreType.DMA,
      ],
      compiler_params=pltpu.CompilerParams(needs_layout_passes=False),
  )
  def k(wt_hbm, rem_hbm, out_hbm, s_a, s_b, o_a, o_b, ia, ib, oa, ob):
    wid = lax.axis_index("s") * _NC + lax.axis_index("c")
    lane = jax.lax.broadcasted_iota(jnp.int32, (16,), 0)

    def blk(step):
      b = wid + step * _NW
      return jnp.where(b < _NBLK, b, b - _NBLK)

    def start_in(b, s_v, sem):
      pltpu.async_copy(wt_hbm.at[:, pl.ds(b * 128, 128)], s_v, sem)

    def wait_in(b, s_v, sem):
      pltpu.make_async_copy(wt_hbm.at[:, pl.ds(b * 128, 128)], s_v,
                            sem).wait()

    def drain_out(o_v, sem):
      pltpu.make_async_copy(o_v, out_hbm.at[pl.ds(0, 2048)], sem).wait()

    start_in(blk(0), s_a, ia)
    start_in(blk(1), s_b, ib)

    def body(it, _):
      b1 = blk(2 * it)
      b2 = blk(2 * it + 1)

      @pl.when(it > 0)
      def _():
        drain_out(o_a, oa)

      wait_in(b1, s_a, ia)
      _shuffle(s_a, o_a, lane, 128)
      pltpu.async_copy(o_a, out_hbm.at[pl.ds(b1 * 2048, 2048)], oa)

      @pl.when(it < _PAIRS - 1)
      def _():
        start_in(blk(2 * it + 2), s_a, ia)

      @pl.when(it > 0)
      def _():
        drain_out(o_b, ob)

      wait_in(b2, s_b, ib)
      _shuffle(s_b, o_b, lane, 128)
      pltpu.async_copy(o_b, out_hbm.at[pl.ds(b2 * 2048, 2048)], ob)

      @pl.when(it < _PAIRS - 1)
      def _():
        start_in(blk(2 * it + 3), s_b, ib)

      return ()

    lax.fori_loop(0, _PAIRS, body, (), unroll=False)
    drain_out(o_a, oa)
    drain_out(o_b, ob)

    @pl.when(wid == _NW - 1)
    def _():
      pltpu.sync_copy(rem_hbm, o_a.at[pl.ds(0, _REM * 16)])
      pltpu.sync_copy(o_a.at[pl.ds(0, _REM * 16)],
                      out_hbm.at[pl.ds(_NBLK * 2048, _REM * 16)])

  return k


def _make_lookup(batch: int):
  assert batch % _NW == 0
  bpw = batch // _NW
  mesh = plsc.VectorSubcoreMesh(core_axis_name="c", subcore_axis_name="s")
  d = _EMBED_DIM

  @functools.partial(
      pl.kernel,
      mesh=mesh,
      out_type=jax.ShapeDtypeStruct((_NUM_FIELDS, d, batch), jnp.float32),
      scratch_types=[
          pltpu.VMEM((bpw,), jnp.int32),
          pltpu.VMEM((bpw, d), jnp.float32),
          pltpu.VMEM((d, bpw), jnp.float32),
          pltpu.SemaphoreType.DMA,
      ],
      compiler_params=pltpu.CompilerParams(use_tc_tiling_on_sc=False,
                                           needs_layout_passes=False),
  )
  def k(idx_hbm, table_hbm, out_hbm, idx_v, rows_v, cols_v, gsem):
    wid = lax.axis_index("s") * _NC + lax.axis_index("c")
    b0 = wid * bpw
    lane = jax.lax.broadcasted_iota(jnp.int32, (16,), 0)

    def body(f, _):
      pltpu.sync_copy(idx_hbm.at[f, pl.ds(b0, bpw)], idx_v)
      pltpu.async_copy(table_hbm.at[idx_v], rows_v, gsem).wait()
      def col(r16, _):
        rows = r16 * 16 + lane
        for j in range(d):
          c = (j + lane) & 15
          val = plsc.load_gather(rows_v, [rows, c])
          plsc.store_scatter(cols_v, [c, rows], val)
        return ()

      lax.fori_loop(0, bpw // 16, col, (), unroll=False)
      pltpu.sync_copy(cols_v, out_hbm.at[f, :, pl.ds(b0, bpw)])
      return ()

    lax.fori_loop(0, _NUM_FIELDS, body, (), unroll=False)

  return k


def kernel(x, weight):
  b, f = x.shape
  offsets = jnp.arange(f, dtype=jnp.int32) * _FIELD_DIM
  idx_t = x.T + offsets[:, None]
  rem_flat = weight[_NBLK * 128:, :].reshape(-1)
  table_flat = _make_retile()(weight.T, rem_flat)
  table = table_flat.reshape(_VOCAB, _EMBED_DIM)
  out_t = _make_lookup(b)(idx_t, table)
  return out_t.transpose(2, 0, 1)


# speedup vs baseline: 3.0023x; 1.1508x over previous
import functools

import jax
import jax.numpy as jnp
from jax import lax
from jax.experimental import pallas as pl
from jax.experimental.pallas import tpu as pltpu
from jax.experimental.pallas import tpu_sc as plsc

_NUM_FIELDS = 26
_FIELD_DIM = 100000
_EMBED_DIM = 16
_VOCAB = _NUM_FIELDS * _FIELD_DIM

_info = plsc.get_sparse_core_info()
_NC, _NS = _info.num_cores, _info.num_subcores
_NW = _NC * _NS

_W = 256
_NBLK = _VOCAB // _W
_REM = _VOCAB - _NBLK * _W
_PAIRS = -(-_NBLK // (2 * _NW))


def _shuffle(s_v, o_v, lane, n):
  def step(i16, _):
    i = i16 * 16 + lane
    for j in range(16):
      d = (j + lane) & 15
      val = plsc.load_gather(s_v, [d, i])
      plsc.store_scatter(o_v, [i * 16 + d], val)
    return ()

  lax.fori_loop(0, n // 16, step, (), unroll=False)


def _make_retile():
  mesh = plsc.VectorSubcoreMesh(core_axis_name="c", subcore_axis_name="s")

  @functools.partial(
      pl.kernel,
      mesh=mesh,
      out_type=jax.ShapeDtypeStruct((_VOCAB * _EMBED_DIM,), jnp.float32),
      scratch_types=[
          pltpu.VMEM((_EMBED_DIM, _W), jnp.float32),
          pltpu.VMEM((_EMBED_DIM, _W), jnp.float32),
          pltpu.VMEM((_W * _EMBED_DIM,), jnp.float32),
          pltpu.VMEM((_W * _EMBED_DIM,), jnp.float32),
          pltpu.SemaphoreType.DMA,
          pltpu.SemaphoreType.DMA,
          pltpu.SemaphoreType.DMA,
          pltpu.SemaphoreType.DMA,
      ],
      compiler_params=pltpu.CompilerParams(needs_layout_passes=False),
  )
  def k(wt_hbm, rem_hbm, out_hbm, s_a, s_b, o_a, o_b, ia, ib, oa, ob):
    wid = lax.axis_index("s") * _NC + lax.axis_index("c")
    lane = jax.lax.broadcasted_iota(jnp.int32, (16,), 0)

    def blk(step):
      b = wid + step * _NW
      return jnp.where(b < _NBLK, b, b - _NBLK)

    def start_in(b, s_v, sem):
      pltpu.async_copy(wt_hbm.at[:, pl.ds(b * _W, _W)], s_v, sem)

    def wait_in(b, s_v, sem):
      pltpu.make_async_copy(wt_hbm.at[:, pl.ds(b * _W, _W)], s_v,
                            sem).wait()

    def drain_out(o_v, sem):
      pltpu.make_async_copy(o_v, out_hbm.at[pl.ds(0, _W * 16)], sem).wait()

    start_in(blk(0), s_a, ia)
    start_in(blk(1), s_b, ib)

    def body(it, _):
      b1 = blk(2 * it)
      b2 = blk(2 * it + 1)

      @pl.when(it > 0)
      def _():
        drain_out(o_a, oa)

      wait_in(b1, s_a, ia)
      _shuffle(s_a, o_a, lane, _W)
      pltpu.async_copy(o_a, out_hbm.at[pl.ds(b1 * (_W * 16), _W * 16)], oa)

      @pl.when(it < _PAIRS - 1)
      def _():
        start_in(blk(2 * it + 2), s_a, ia)

      @pl.when(it > 0)
      def _():
        drain_out(o_b, ob)

      wait_in(b2, s_b, ib)
      _shuffle(s_b, o_b, lane, _W)
      pltpu.async_copy(o_b, out_hbm.at[pl.ds(b2 * (_W * 16), _W * 16)], ob)

      @pl.when(it < _PAIRS - 1)
      def _():
        start_in(blk(2 * it + 3), s_b, ib)

      return ()

    lax.fori_loop(0, _PAIRS, body, (), unroll=False)
    drain_out(o_a, oa)
    drain_out(o_b, ob)

    @pl.when(wid == _NW - 1)
    def _():
      pltpu.sync_copy(rem_hbm, o_a.at[pl.ds(0, _REM * 16)])
      pltpu.sync_copy(o_a.at[pl.ds(0, _REM * 16)],
                      out_hbm.at[pl.ds(_NBLK * (_W * 16), _REM * 16)])

  return k


def _make_lookup(batch: int):
  assert batch % _NW == 0
  bpw = batch // _NW
  mesh = plsc.VectorSubcoreMesh(core_axis_name="c", subcore_axis_name="s")
  d = _EMBED_DIM

  @functools.partial(
      pl.kernel,
      mesh=mesh,
      out_type=jax.ShapeDtypeStruct((_NUM_FIELDS, d, batch), jnp.float32),
      scratch_types=[
          pltpu.VMEM((bpw,), jnp.int32),
          pltpu.VMEM((bpw, d), jnp.float32),
          pltpu.VMEM((d, bpw), jnp.float32),
          pltpu.SemaphoreType.DMA,
      ],
      compiler_params=pltpu.CompilerParams(use_tc_tiling_on_sc=False,
                                           needs_layout_passes=False),
  )
  def k(idx_hbm, table_hbm, out_hbm, idx_v, rows_v, cols_v, gsem):
    wid = lax.axis_index("s") * _NC + lax.axis_index("c")
    b0 = wid * bpw
    lane = jax.lax.broadcasted_iota(jnp.int32, (16,), 0)

    def body(f, _):
      pltpu.sync_copy(idx_hbm.at[f, pl.ds(b0, bpw)], idx_v)
      pltpu.async_copy(table_hbm.at[idx_v], rows_v, gsem).wait()
      def col(r16, _):
        rows = r16 * 16 + lane
        for j in range(d):
          c = (j + lane) & 15
          val = plsc.load_gather(rows_v, [rows, c])
          plsc.store_scatter(cols_v, [c, rows], val)
        return ()

      lax.fori_loop(0, bpw // 16, col, (), unroll=False)
      pltpu.sync_copy(cols_v, out_hbm.at[f, :, pl.ds(b0, bpw)])
      return ()

    lax.fori_loop(0, _NUM_FIELDS, body, (), unroll=False)

  return k


def kernel(x, weight):
  b, f = x.shape
  offsets = jnp.arange(f, dtype=jnp.int32) * _FIELD_DIM
  idx_t = x.T + offsets[:, None]
  rem_flat = weight[_NBLK * _W:, :].reshape(-1)
  table_flat = _make_retile()(weight.T, rem_flat)
  table = table_flat.reshape(_VOCAB, _EMBED_DIM)
  out_t = _make_lookup(b)(idx_t, table)
  return out_t.transpose(2, 0, 1)
